# baseline (speedup 1.0000x reference)
import functools

import jax
import jax.numpy as jnp
from jax import lax
from jax.experimental import pallas as pl
from jax.experimental.pallas import tpu as pltpu
from jax.experimental.pallas import tpu_sc as plsc

B, D, H, W = 2, 4, 384, 384
NW = 32
RPW = (B * H) // NW
COFF = 16
PW = W + 2 * COFF
BONUS = 10.0

@functools.cache
def _build_sc_kernel():
    mesh = plsc.VectorSubcoreMesh(core_axis_name="c", subcore_axis_name="s")
    return functools.partial(
        pl.kernel,
        mesh=mesh,
        compiler_params=pltpu.CompilerParams(use_tc_tiling_on_sc=False, needs_layout_passes=False),
        out_type=[
            jax.ShapeDtypeStruct((B, 3, D, H, W), jnp.float32),
            jax.ShapeDtypeStruct((B, D, H, W), jnp.float32),
        ],
        scratch_types=[
            pltpu.VMEM((D, RPW + 2, PW), jnp.float32),
            pltpu.VMEM((RPW, W), jnp.float32),
            pltpu.VMEM((RPW, W), jnp.float32),
            pltpu.VMEM((RPW, W), jnp.float32),
            pltpu.VMEM((RPW, W), jnp.float32),
        ],
    )(_sc_body)


def _sc_body(x_hbm, c_hbm, y_hbm, xpad, yv, c0v, c1v, c2v):
    cid = lax.axis_index("c")
    sid = lax.axis_index("s")
    wid = cid * 16 + sid
    b = wid // 16
    r0 = (wid % 16) * RPW

    iota_i = lax.iota(jnp.int32, 16)
    iota_f = iota_i.astype(jnp.float32)

    row_n = jnp.maximum(r0 - 1, 0)
    row_s = jnp.minimum(r0 + RPW, H - 1)
    for d in range(D):
        pltpu.sync_copy(x_hbm.at[b, d, pl.ds(row_n, 1)],
                        xpad.at[d, 0:1, COFF:COFF + W])
        pltpu.sync_copy(x_hbm.at[b, d, pl.ds(r0, RPW)],
                        xpad.at[d, 1:1 + RPW, COFF:COFF + W])
        pltpu.sync_copy(x_hbm.at[b, d, pl.ds(row_s, 1)],
                        xpad.at[d, 1 + RPW:2 + RPW, COFF:COFF + W])

    m_first = iota_i == 0
    m_last = iota_i == 15

    def d_body(d, carry):
        dm = jnp.maximum(d - 1, 0)
        dp = jnp.minimum(d + 1, D - 1)
        d_f = d.astype(jnp.float32)

        def h_body(h, carry2):
            rn, rc, rs = h, h + 1, h + 2
            h_f = (r0 + h).astype(jnp.float32)
            for j in range(W // 16):
                col = COFF + j * 16

                def ld3(pd, r):
                    c = xpad[pd, r, pl.ds(col, 16)]
                    wv = xpad[pd, r, pl.ds(col - 1, 16)]
                    ev = xpad[pd, r, pl.ds(col + 1, 16)]
                    if j == 0:
                        wv = jnp.where(m_first, c, wv)
                    if j == W // 16 - 1:
                        ev = jnp.where(m_last, c, ev)
                    return wv, c, ev

                pc_nw, pc_nc, pc_ne = ld3(d, rn)
                pc_cw, x0, pc_ce = ld3(d, rc)
                pc_sw, pc_sc, pc_se = ld3(d, rs)
                pm_nw, pm_nc, pm_ne = ld3(dm, rn)
                pm_cw, pm_cc, pm_ce = ld3(dm, rc)
                pm_sw, pm_sc, pm_se = ld3(dm, rs)
                pp_nw, pp_nc, pp_ne = ld3(dp, rn)
                pp_cw, pp_cc, pp_ce = ld3(dp, rc)
                pp_sw, pp_sc, pp_se = ld3(dp, rs)

                mx = jnp.maximum(pc_nw, pc_nc)
                mx = jnp.maximum(mx, pc_ne)
                mx = jnp.maximum(mx, pc_cw)
                mx = jnp.maximum(mx, pc_ce)
                mx = jnp.maximum(mx, pc_sw)
                mx = jnp.maximum(mx, pc_sc)
                mx = jnp.maximum(mx, pc_se)
                for v in (pm_nw, pm_nc, pm_ne, pm_cw, pm_cc, pm_ce,
                          pm_sw, pm_sc, pm_se,
                          pp_nw, pp_nc, pp_ne, pp_cw, pp_cc, pp_ce,
                          pp_sw, pp_sc, pp_se):
                    mx = jnp.maximum(mx, v)
                nms = x0 > mx

                gx = 0.5 * (pc_ce - pc_cw)
                gy = 0.5 * (pc_sc - pc_nc)
                gs = 0.5 * (pp_cc - pm_cc)
                dxx = pc_ce + pc_cw - 2.0 * x0
                dyy = pc_sc + pc_nc - 2.0 * x0
                dss = pp_cc + pm_cc - 2.0 * x0
                dxy = 0.25 * (pc_nw - pc_ne - pc_sw + pc_se)
                dys = 0.25 * (pm_nc - pm_sc - pp_nc + pp_sc)
                dxs = 0.25 * (pm_cw - pm_ce - pp_cw + pp_ce)

                c00 = dyy * dss - dys * dys
                c01 = dxy * dss - dys * dxs
                c02 = dxy * dys - dyy * dxs
                det = dxx * c00 - dxy * c01 + dxs * c02
                ok = jnp.abs(det) > 0.0

                inv_det = 1.0 / det
                a01 = -c01
                a11 = dxx * dss - dxs * dxs
                a12 = dxy * dxs - dxx * dys
                a22 = dxx * dyy - dxy * dxy
                x_sol = (c00 * gx + a01 * gy + c02 * gs) * inv_det
                y_sol = (a01 * gx + a11 * gy + a12 * gs) * inv_det
                s_sol = (c02 * gx + a12 * gy + a22 * gs) * inv_det

                amax = jnp.maximum(jnp.maximum(jnp.abs(x_sol),
                                               jnp.abs(y_sol)),
                                   jnp.abs(s_sol))
                new_nms = nms & ok & (amax < jnp.inf)
                keep = new_nms & (amax <= 0.7)

                zero = jnp.zeros_like(x0)
                dx0 = jnp.where(keep, -x_sol, zero)
                dx1 = jnp.where(keep, -y_sol, zero)
                dx2 = jnp.where(keep, -s_sol, zero)

                dy = 0.5 * (gx * dx0 + gy * dx1 + gs * dx2)
                yv[h, pl.ds(j * 16, 16)] = (
                    x0 + dy + BONUS * new_nms.astype(jnp.float32))
                c0v[h, pl.ds(j * 16, 16)] = d_f + dx2
                c1v[h, pl.ds(j * 16, 16)] = (
                    float(j * 16) + iota_f + dx0)
                c2v[h, pl.ds(j * 16, 16)] = h_f + dx1
            return carry2

        lax.fori_loop(0, RPW, h_body, 0)

        pltpu.sync_copy(yv, y_hbm.at[b, d, pl.ds(r0, RPW)])
        pltpu.sync_copy(c0v, c_hbm.at[b, 0, d, pl.ds(r0, RPW)])
        pltpu.sync_copy(c1v, c_hbm.at[b, 1, d, pl.ds(r0, RPW)])
        pltpu.sync_copy(c2v, c_hbm.at[b, 2, d, pl.ds(r0, RPW)])
        return carry

    lax.fori_loop(0, D, d_body, 0)


def kernel(x):
    xs = x.reshape(B, D, H, W)
    coords, y = _build_sc_kernel()(xs)
    return (coords.reshape(B, 1, 3, D, H, W), y.reshape(B, 1, D, H, W))


# speedup vs baseline: 7.0338x; 7.0338x over previous
import jax
import jax.numpy as jnp
from jax import lax
from jax.experimental import pallas as pl
from jax.experimental.pallas import tpu as pltpu

B, D, H, W = 2, 4, 384, 384
T = 4
HB = H // T
PR = HB + 16
BONUS = 10.0


def _kern(xp_ref, xc_ref, xn_ref, y_ref, c_ref, xs_ref, emw_ref, w3_ref):
    t = pl.program_id(1)
    first = t == 0
    last = t == T - 1

    planes, a_c, b_c = [], [], []
    for d in range(D):
        x0 = xc_ref[0, d, 0]
        planes.append(x0)
        above = jnp.where(first, x0[:1], xp_ref[0, d, 0, HB - 1:HB])
        below = jnp.where(last, x0[HB - 1:HB], xn_ref[0, d, 0, :1])
        xs_ref[d, 8:8 + HB] = x0
        xs_ref[d, 7:8] = above
        xs_ref[d, 8 + HB:9 + HB] = below

    for d in range(D):
        xv = xs_ref[d]
        a = jnp.concatenate([xv[:, :1], xv[:, :W - 1]], axis=1)
        b = jnp.concatenate([xv[:, 1:], xv[:, W - 1:]], axis=1)
        emw_ref[d] = b - a
        w3_ref[d] = jnp.maximum(jnp.maximum(a, xv), b)
        a_c.append(a[8:8 + HB])
        b_c.append(b[8:8 + HB])

    m9, ring8 = [], []
    for d in range(D):
        w3n = w3_ref[d, 7:7 + HB]
        w3c = w3_ref[d, 8:8 + HB]
        w3s = w3_ref[d, 9:9 + HB]
        m9.append(jnp.maximum(jnp.maximum(w3n, w3c), w3s))
        lr = jnp.maximum(a_c[d], b_c[d])
        ring8.append(jnp.maximum(jnp.maximum(w3n, w3s), lr))

    iota_w = lax.broadcasted_iota(jnp.int32, (HB, W), 1).astype(jnp.float32)
    iota_h = (lax.broadcasted_iota(jnp.int32, (HB, W), 0)
              + t * HB).astype(jnp.float32)

    for d in range(D):
        dm, dp = max(d - 1, 0), min(d + 1, D - 1)
        x0 = planes[d]
        mx = jnp.maximum(jnp.maximum(m9[dm], m9[dp]), ring8[d])
        nms = x0 > mx

        n_c = xs_ref[d, 7:7 + HB]
        s_c = xs_ref[d, 9:9 + HB]
        emw_c = emw_ref[d, 8:8 + HB]

        gx = 0.5 * emw_c
        gy = 0.5 * (s_c - n_c)
        gs = 0.5 * (planes[dp] - planes[dm])

        dxx = a_c[d] + b_c[d] - 2.0 * x0
        dyy = s_c + n_c - 2.0 * x0
        dss = planes[dp] + planes[dm] - 2.0 * x0
        dxy = 0.25 * (emw_ref[d, 9:9 + HB] - emw_ref[d, 7:7 + HB])
        dys = 0.25 * (xs_ref[dm, 7:7 + HB] - xs_ref[dm, 9:9 + HB]
                      - xs_ref[dp, 7:7 + HB] + xs_ref[dp, 9:9 + HB])
        dxs = 0.25 * (emw_ref[dp, 8:8 + HB] - emw_ref[dm, 8:8 + HB])

        c00 = dyy * dss - dys * dys
        c01 = dxy * dss - dys * dxs
        c02 = dxy * dys - dyy * dxs
        det = dxx * c00 - dxy * c01 + dxs * c02
        ok = jnp.abs(det) > 0.0

        inv_det = 1.0 / det
        a11 = dxx * dss - dxs * dxs
        a12 = dxy * dxs - dxx * dys
        a22 = dxx * dyy - dxy * dxy
        x_sol = (c00 * gx - c01 * gy + c02 * gs) * inv_det
        y_sol = (a11 * gy - c01 * gx + a12 * gs) * inv_det
        s_sol = (c02 * gx + a12 * gy + a22 * gs) * inv_det

        amax = jnp.maximum(jnp.maximum(jnp.abs(x_sol), jnp.abs(y_sol)),
                           jnp.abs(s_sol))
        new_nms = nms & ok & (amax < jnp.inf)
        keep = new_nms & (amax <= 0.7)

        zero = jnp.zeros_like(x0)
        dx0 = jnp.where(keep, -x_sol, zero)
        dx1 = jnp.where(keep, -y_sol, zero)
        dx2 = jnp.where(keep, -s_sol, zero)

        dy = 0.5 * (gx * dx0 + gy * dx1 + gs * dx2)
        y_ref[0, d, 0] = x0 + dy + BONUS * new_nms.astype(jnp.float32)

        c_ref[0, 0, d, 0] = float(d) + dx2
        c_ref[0, 1, d, 0] = iota_w + dx0
        c_ref[0, 2, d, 0] = iota_h + dx1


def kernel(x):
    xt = x.reshape(B, D, T, HB, W)

    def mk_spec(off):
        return pl.BlockSpec(
            (1, D, 1, HB, W),
            lambda b, t: (b, 0, jnp.clip(t + off, 0, T - 1), 0, 0))

    y, coords = pl.pallas_call(
        _kern,
        grid=(B, T),
        in_specs=[mk_spec(-1), mk_spec(0), mk_spec(1)],
        out_specs=[
            pl.BlockSpec((1, D, 1, HB, W), lambda b, t: (b, 0, t, 0, 0)),
            pl.BlockSpec((1, 3, D, 1, HB, W),
                         lambda b, t: (b, 0, 0, t, 0, 0)),
        ],
        out_shape=[
            jax.ShapeDtypeStruct((B, D, T, HB, W), jnp.float32),
            jax.ShapeDtypeStruct((B, 3, D, T, HB, W), jnp.float32),
        ],
        scratch_shapes=[
            pltpu.VMEM((D, PR, W), jnp.float32),
            pltpu.VMEM((D, PR, W), jnp.float32),
            pltpu.VMEM((D, PR, W), jnp.float32),
        ],
        compiler_params=pltpu.CompilerParams(
            dimension_semantics=("parallel", "parallel")),
    )(xt, xt, xt)
    coords = coords.reshape(B, 1, 3, D, H, W)
    y = y.reshape(B, 1, D, H, W)
    return coords, y
